# baseline (device time: 31148 ns/iter reference)
import jax
import jax.numpy as jnp
from jax import lax
from jax.experimental import pallas as pl
from jax.experimental.pallas import tpu as pltpu

B, SQ, H, D = 4, 32, 8, 128
N_SPLIT = 4
BLK = 4096 // N_SPLIT
SCALE = D ** -0.5
NR = 3


def _fused_body(q_ref, k_hbm, v_hbm, out_ref,
                kbuf, vbuf, acc_ref, accl_ref,
                sb_ref, sbl_ref, rb_ref, rbl_ref,
                dma_sems, snum_sem, rnum_sem, sl_sem, rl_sem):
    mx = lax.axis_index("x")
    my = lax.axis_index("y")
    mz = lax.axis_index("z")
    start = (my * 2 + mz) * BLK
    peers = [(mx, my, 1 - mz), (mx, 1 - my, mz), (1 - mx, my, mz)]

    barrier = pltpu.get_barrier_semaphore()
    for p in peers:
        pl.semaphore_signal(barrier, inc=1, device_id=p,
                            device_id_type=pl.DeviceIdType.MESH)
    pl.semaphore_wait(barrier, 3)

    def start_dma(b):
        slot = b % 2
        ck = pltpu.make_async_copy(
            k_hbm.at[b, pl.ds(start, BLK)], kbuf.at[slot], dma_sems.at[0, slot])
        cv = pltpu.make_async_copy(
            v_hbm.at[b, pl.ds(start, BLK)], vbuf.at[slot], dma_sems.at[1, slot])
        ck.start()
        cv.start()
        return ck, cv

    def make_rdma(b, r):
        cn = pltpu.make_async_remote_copy(
            src_ref=sb_ref.at[b, r], dst_ref=rb_ref.at[b, r],
            send_sem=snum_sem.at[b, r], recv_sem=rnum_sem.at[b, r],
            device_id=peers[r], device_id_type=pl.DeviceIdType.MESH)
        cl = pltpu.make_async_remote_copy(
            src_ref=sbl_ref.at[b, r], dst_ref=rbl_ref.at[b, r],
            send_sem=sl_sem.at[b, r], recv_sem=rl_sem.at[b, r],
            device_id=peers[r], device_id_type=pl.DeviceIdType.MESH)
        return cn, cl

    dmas = {0: start_dma(0)}
    rdmas = {}

    for s in range(B + NR):
        if s < B:
            if s + 1 < B:
                dmas[s + 1] = start_dma(s + 1)
            ck, cv = dmas.pop(s)
            ck.wait()
            cv.wait()
            slot = s % 2
            q3 = (q_ref[s] * SCALE).astype(jnp.bfloat16)
            kt = jnp.swapaxes(kbuf[slot].astype(jnp.bfloat16), 0, 1)
            vt = jnp.swapaxes(vbuf[slot].astype(jnp.bfloat16), 0, 1)
            ones = jnp.ones((BLK, 1), jnp.bfloat16)
            ls = []
            for h in range(H):
                sc = lax.dot_general(
                    q3[:, h, :], kt[h], (((1,), (1,)), ((), ())),
                    preferred_element_type=jnp.float32)
                e = jnp.exp(sc.astype(jnp.bfloat16))
                ls.append(lax.dot_general(
                    e, ones, (((1,), (0,)), ((), ())),
                    preferred_element_type=jnp.float32))
                acc_ref[s, :, h, :] = lax.dot_general(
                    e, vt[h], (((1,), (0,)), ((), ())),
                    preferred_element_type=jnp.float32)
            accl_ref[s] = jnp.concatenate(ls, axis=1)
            sb_ref[s, 0] = acc_ref[s].astype(jnp.bfloat16)
            sbl_ref[s, 0] = accl_ref[s]
            cn, cl = make_rdma(s, 0)
            cn.start()
            cl.start()
            rdmas[(s, 0)] = (cn, cl)

        for r in range(NR):
            b = s - 1 - r
            if not (0 <= b < B):
                continue
            cn, cl = rdmas[(b, r)]
            cn.wait_recv()
            cl.wait_recv()
            acc_ref[b] += rb_ref[b, r].astype(jnp.float32)
            accl_ref[b] += rbl_ref[b, r]
            if r + 1 < NR:
                sb_ref[b, r + 1] = acc_ref[b].astype(jnp.bfloat16)
                sbl_ref[b, r + 1] = accl_ref[b]
                cn2, cl2 = make_rdma(b, r + 1)
                cn2.start()
                cl2.start()
                rdmas[(b, r + 1)] = (cn2, cl2)
            else:
                out_ref[b] = acc_ref[b] / accl_ref[b][:, :, None]

    for cn, cl in rdmas.values():
        cn.wait_send()
        cl.wait_send()


def kernel(Q, K, V):
    return pl.pallas_call(
        _fused_body,
        out_shape=jax.ShapeDtypeStruct((B, SQ, H, D), jnp.float32),
        in_specs=[
            pl.BlockSpec(memory_space=pltpu.VMEM),
            pl.BlockSpec(memory_space=pl.ANY),
            pl.BlockSpec(memory_space=pl.ANY),
        ],
        out_specs=pl.BlockSpec(memory_space=pltpu.VMEM),
        scratch_shapes=[
            pltpu.VMEM((2, BLK, H, D), jnp.float32),
            pltpu.VMEM((2, BLK, H, D), jnp.float32),
            pltpu.VMEM((B, SQ, H, D), jnp.float32),
            pltpu.VMEM((B, SQ, H), jnp.float32),
            pltpu.VMEM((B, NR, SQ, H, D), jnp.bfloat16),
            pltpu.VMEM((B, NR, SQ, H), jnp.float32),
            pltpu.VMEM((B, NR, SQ, H, D), jnp.bfloat16),
            pltpu.VMEM((B, NR, SQ, H), jnp.float32),
            pltpu.SemaphoreType.DMA((2, 2)),
            pltpu.SemaphoreType.DMA((B, NR)),
            pltpu.SemaphoreType.DMA((B, NR)),
            pltpu.SemaphoreType.DMA((B, NR)),
            pltpu.SemaphoreType.DMA((B, NR)),
        ],
        compiler_params=pltpu.CompilerParams(
            collective_id=0,
            vmem_limit_bytes=100 * 1024 * 1024,
        ),
    )(Q, K, V)


# device time: 16553 ns/iter; 1.8817x vs baseline; 1.8817x over previous
import os

import jax
import jax.numpy as jnp
from jax import lax
from jax.experimental import pallas as pl
from jax.experimental.pallas import tpu as pltpu

B, SQ, H, D = 4, 32, 8, 128
N_SPLIT = 4
BLK = 4096 // N_SPLIT
SCALE = D ** -0.5
NR = 3
_ABLATE_COMM = os.environ.get("ABLATE_COMM") == "1"


def _fused_body(q_ref, k_hbm, v_hbm, out_ref,
                kbuf, vbuf, acc_ref, accl_ref,
                sb_ref, sbl_ref, rb_ref, rbl_ref,
                dma_sems, snum_sem, rnum_sem, sl_sem, rl_sem):
    mx = lax.axis_index("x")
    my = lax.axis_index("y")
    mz = lax.axis_index("z")
    start = (my * 2 + mz) * BLK
    peers = [(mx, my, 1 - mz), (mx, 1 - my, mz), (1 - mx, my, mz)]

    if not _ABLATE_COMM:
        barrier = pltpu.get_barrier_semaphore()
        for p in peers:
            pl.semaphore_signal(barrier, inc=1, device_id=p,
                                device_id_type=pl.DeviceIdType.MESH)
        pl.semaphore_wait(barrier, 3)

    def start_dma(b):
        slot = b % 2
        ck = pltpu.make_async_copy(
            k_hbm.at[b, pl.ds(start, BLK)], kbuf.at[slot], dma_sems.at[0, slot])
        cv = pltpu.make_async_copy(
            v_hbm.at[b, pl.ds(start, BLK)], vbuf.at[slot], dma_sems.at[1, slot])
        ck.start()
        cv.start()
        return ck, cv

    def make_rdma(b, r):
        cn = pltpu.make_async_remote_copy(
            src_ref=sb_ref.at[b, r], dst_ref=rb_ref.at[b, r],
            send_sem=snum_sem.at[b, r], recv_sem=rnum_sem.at[b, r],
            device_id=peers[r], device_id_type=pl.DeviceIdType.MESH)
        cl = pltpu.make_async_remote_copy(
            src_ref=sbl_ref.at[b, r], dst_ref=rbl_ref.at[b, r],
            send_sem=sl_sem.at[b, r], recv_sem=rl_sem.at[b, r],
            device_id=peers[r], device_id_type=pl.DeviceIdType.MESH)
        return cn, cl

    dmas = {0: start_dma(0)}
    rdmas = {}

    for s in range(B + NR):
        if s < B:
            if s + 1 < B:
                dmas[s + 1] = start_dma(s + 1)
            ck, cv = dmas.pop(s)
            ck.wait()
            cv.wait()
            slot = s % 2
            q3 = (q_ref[s] * SCALE).astype(jnp.bfloat16)
            kt = jnp.swapaxes(kbuf[slot].astype(jnp.bfloat16), 0, 1)
            vt = jnp.swapaxes(vbuf[slot].astype(jnp.bfloat16), 0, 1)
            ls = []
            for h in range(H):
                sc = lax.dot_general(
                    q3[:, h, :], kt[h], (((1,), (1,)), ((), ())),
                    preferred_element_type=jnp.float32)
                e = jnp.exp(sc)
                ls.append(jnp.sum(e, axis=1, keepdims=True))
                acc_ref[s, :, h, :] = lax.dot_general(
                    e.astype(jnp.bfloat16), vt[h], (((1,), (0,)), ((), ())),
                    preferred_element_type=jnp.float32)
            accl_ref[s] = jnp.concatenate(ls, axis=1)
            if _ABLATE_COMM:
                out_ref[s] = acc_ref[s] / accl_ref[s][:, :, None]
                continue
            sb_ref[s, 0] = acc_ref[s].astype(jnp.bfloat16)
            sbl_ref[s, 0] = accl_ref[s]
            cn, cl = make_rdma(s, 0)
            cn.start()
            cl.start()
            rdmas[(s, 0)] = (cn, cl)

        if _ABLATE_COMM:
            continue
        for r in range(NR):
            b = s - 1 - r
            if not (0 <= b < B):
                continue
            cn, cl = rdmas[(b, r)]
            cn.wait_recv()
            cl.wait_recv()
            acc_ref[b] += rb_ref[b, r].astype(jnp.float32)
            accl_ref[b] += rbl_ref[b, r]
            if r + 1 < NR:
                sb_ref[b, r + 1] = acc_ref[b].astype(jnp.bfloat16)
                sbl_ref[b, r + 1] = accl_ref[b]
                cn2, cl2 = make_rdma(b, r + 1)
                cn2.start()
                cl2.start()
                rdmas[(b, r + 1)] = (cn2, cl2)
            else:
                out_ref[b] = acc_ref[b] / accl_ref[b][:, :, None]

    for cn, cl in rdmas.values():
        cn.wait_send()
        cl.wait_send()


def kernel(Q, K, V):
    return pl.pallas_call(
        _fused_body,
        out_shape=jax.ShapeDtypeStruct((B, SQ, H, D), jnp.float32),
        in_specs=[
            pl.BlockSpec(memory_space=pltpu.VMEM),
            pl.BlockSpec(memory_space=pl.ANY),
            pl.BlockSpec(memory_space=pl.ANY),
        ],
        out_specs=pl.BlockSpec(memory_space=pltpu.VMEM),
        scratch_shapes=[
            pltpu.VMEM((2, BLK, H, D), jnp.float32),
            pltpu.VMEM((2, BLK, H, D), jnp.float32),
            pltpu.VMEM((B, SQ, H, D), jnp.float32),
            pltpu.VMEM((B, SQ, H), jnp.float32),
            pltpu.VMEM((B, NR, SQ, H, D), jnp.bfloat16),
            pltpu.VMEM((B, NR, SQ, H), jnp.float32),
            pltpu.VMEM((B, NR, SQ, H, D), jnp.bfloat16),
            pltpu.VMEM((B, NR, SQ, H), jnp.float32),
            pltpu.SemaphoreType.DMA((2, 2)),
            pltpu.SemaphoreType.DMA((B, NR)),
            pltpu.SemaphoreType.DMA((B, NR)),
            pltpu.SemaphoreType.DMA((B, NR)),
            pltpu.SemaphoreType.DMA((B, NR)),
        ],
        compiler_params=pltpu.CompilerParams(
            collective_id=None if _ABLATE_COMM else 0,
            vmem_limit_bytes=100 * 1024 * 1024,
        ),
    )(Q, K, V)
